# baseline (speedup 1.0000x reference)
import functools

import jax
import jax.numpy as jnp
from jax import lax
from jax.experimental import pallas as pl
from jax.experimental.pallas import tpu as pltpu
from jax.experimental.pallas import tpu_sc as plsc

N = 8192
HIDDEN = 768
REL = 53
B = 512
LANES = 128
LSE_COL = REL
NUM_CORES = 2
NUM_SUBCORES = 16
NUM_WORKERS = NUM_CORES * NUM_SUBCORES
BAGS_PER_W = B // NUM_WORKERS
CHUNK = 128
ROW_BLK = 4096
QROWS = 64
NEG = -3.0e38
IMAX = 2**31 - 1


def _tc_body(x_ref, w_ref, b_ref, a_ref, pt_ref):
    xb = x_ref[...]
    logits = lax.dot_general(
        xb, w_ref[...], (((1,), (0,)), ((), ())),
        precision=None,
        preferred_element_type=jnp.float32,
    ) + b_ref[...]
    m = jnp.max(logits, axis=1, keepdims=True)
    u = jnp.exp(logits - m)
    probs = u / jnp.sum(u, axis=1, keepdims=True)
    a_ref[...] = logits
    for k in range(ROW_BLK // 128):
        pt_ref[:, k, :] = probs[k * 128:(k + 1) * 128, :QROWS].T


def _sc_body(pt_hbm, a_hbm, meta_hbm, out_hbm,
             meta_v, idx_v, blocks_v, js_v, outrows_v, vb_ref, va_ref, sem):
    wid = lax.axis_index("s") * NUM_CORES + lax.axis_index("c")
    base = wid * BAGS_PER_W
    pltpu.sync_copy(meta_hbm.at[wid], meta_v)
    lane = lax.broadcasted_iota(jnp.int32, (16,), 0)
    a_vec = meta_v[pl.ds(0, 16)]
    e_vec = meta_v[pl.ds(16, 16)]
    q_vec = meta_v[pl.ds(32, 16)]

    blk0 = a_vec >> 7
    last = (e_vec - 1) >> 7
    nblk = last - blk0 + 1
    rounds = nblk[0]
    for l in range(1, 16):
        rounds = jnp.maximum(rounds, nblk[l])

    neg16 = jnp.full((16,), NEG, jnp.float32)
    zero16 = jnp.zeros((16,), jnp.int32)
    for b in range(BAGS_PER_W):
        vb_ref[pl.ds(b * 16, 16)] = neg16
        va_ref[pl.ds(b * 16, 16)] = zero16

    def round_body(r, carry):
        blk = jnp.minimum(blk0 + r, last)
        idx_v[...] = q_vec * (N // 128) + blk
        pltpu.async_copy(pt_hbm.at[idx_v], blocks_v, sem).wait()
        for b in range(BAGS_PER_W):
            rowbase = blk[b] * 128
            a_s = a_vec[b]
            e_s = e_vec[b]
            vb = vb_ref[pl.ds(b * 16, 16)]
            va = va_ref[pl.ds(b * 16, 16)]
            for j in range(8):
                sv = blocks_v[b, pl.ds(j * 16, 16)]
                rid = rowbase + j * 16 + lane
                ok = (rid >= a_s) & (rid < e_s)
                val = jnp.where(ok, sv, NEG)
                upd = val > vb
                vb = jnp.where(upd, val, vb)
                va = jnp.where(upd, rid, va)
            vb_ref[pl.ds(b * 16, 16)] = vb
            va_ref[pl.ds(b * 16, 16)] = va
        return carry

    lax.fori_loop(0, rounds, round_body, jnp.int32(0))

    jsv = zero16
    for b in range(BAGS_PER_W):
        vb = vb_ref[pl.ds(b * 16, 16)]
        va = va_ref[pl.ds(b * 16, 16)]
        best = vb[0]
        barg = va[0]
        for l in range(1, 16):
            k = vb[l]
            r = va[l]
            upd = (k > best) | ((k == best) & (r < barg))
            best = jnp.where(upd, k, best)
            barg = jnp.where(upd, r, barg)
        jsv = jnp.where(lane == b, barg, jsv)

    js_v[...] = jsv
    pltpu.async_copy(a_hbm.at[js_v], outrows_v, sem).wait()
    pltpu.sync_copy(outrows_v, out_hbm.at[pl.ds(base, BAGS_PER_W)])


@functools.lru_cache(maxsize=1)
def _sc_select():
    return pl.kernel(
        _sc_body,
        out_type=jax.ShapeDtypeStruct((B, LANES), jnp.float32),
        mesh=plsc.VectorSubcoreMesh(core_axis_name="c", subcore_axis_name="s",
                                    num_cores=NUM_CORES,
                                    num_subcores=NUM_SUBCORES),
        scratch_types=[
            pltpu.VMEM((3 * BAGS_PER_W,), jnp.int32),
            pltpu.VMEM((BAGS_PER_W,), jnp.int32),
            pltpu.VMEM((BAGS_PER_W, 128), jnp.float32),
            pltpu.VMEM((BAGS_PER_W,), jnp.int32),
            pltpu.VMEM((BAGS_PER_W, LANES), jnp.float32),
            pltpu.VMEM((BAGS_PER_W * 16,), jnp.float32),
            pltpu.VMEM((BAGS_PER_W * 16,), jnp.int32),
            pltpu.SemaphoreType.DMA,
        ],
    )


def _tc_logits(x, w_pad, bias_pad):
    return pl.pallas_call(
        _tc_body,
        grid=(N // ROW_BLK,),
        in_specs=[
            pl.BlockSpec((ROW_BLK, HIDDEN), lambda i: (i, 0)),
            pl.BlockSpec((HIDDEN, LANES), lambda i: (0, 0)),
            pl.BlockSpec((1, LANES), lambda i: (0, 0)),
        ],
        out_specs=[
            pl.BlockSpec((ROW_BLK, LANES), lambda i: (i, 0)),
            pl.BlockSpec((QROWS, ROW_BLK // 128, 128), lambda i: (0, i, 0)),
        ],
        out_shape=[
            jax.ShapeDtypeStruct((N, LANES), jnp.float32),
            jax.ShapeDtypeStruct((QROWS, N // 128, 128), jnp.float32),
        ],
    )(x, w_pad, bias_pad)


def kernel(x, scope, query, rel_mat, bias):
    w_pad = jnp.zeros((HIDDEN, LANES), jnp.float32).at[:, :REL].set(rel_mat)
    bias_pad = jnp.full((LANES,), -1e30, jnp.float32).at[:REL].set(bias)
    A, PT = _tc_logits(x, w_pad, bias_pad.reshape(1, LANES))
    scope32 = scope.astype(jnp.int32)
    meta = jnp.concatenate([
        scope32[:, 0].reshape(NUM_WORKERS, BAGS_PER_W),
        scope32[:, 1].reshape(NUM_WORKERS, BAGS_PER_W),
        query.astype(jnp.int32).reshape(NUM_WORKERS, BAGS_PER_W),
    ], axis=1)
    out_full = _sc_select()(PT.reshape(QROWS * (N // 128), 128), A, meta)
    return out_full[:, :REL], rel_mat


# speedup vs baseline: 1.0092x; 1.0092x over previous
import functools

import jax
import jax.numpy as jnp
from jax import lax
from jax.experimental import pallas as pl
from jax.experimental.pallas import tpu as pltpu
from jax.experimental.pallas import tpu_sc as plsc

N = 8192
HIDDEN = 768
REL = 53
B = 512
LANES = 128
NUM_CORES = 2
NUM_SUBCORES = 16
NUM_WORKERS = NUM_CORES * NUM_SUBCORES
BAGS_PER_W = B // NUM_WORKERS
ROW_BLK = 2048
QROWS = 64
NEG = -3.0e38


def _tc_body(x_ref, w_ref, b_ref, a_ref, pt_ref):
    xb = x_ref[...]
    logits = lax.dot_general(
        xb, w_ref[...], (((1,), (0,)), ((), ())),
        precision=None,
        preferred_element_type=jnp.float32,
    ) + b_ref[...]
    m = jnp.max(logits, axis=1, keepdims=True)
    u = jnp.exp(logits - m)
    probs = u / jnp.sum(u, axis=1, keepdims=True)
    a_ref[...] = logits
    for k in range(ROW_BLK // 128):
        pt_ref[:, k, :] = probs[k * 128:(k + 1) * 128, :QROWS].T


def _sc_body(pt_hbm, a_hbm, meta_hbm, out_hbm,
             meta_v, blocks_v, outrows_v, vb_ref, va_ref, sem):
    wid = lax.axis_index("s") * NUM_CORES + lax.axis_index("c")
    base = wid * BAGS_PER_W
    pltpu.sync_copy(meta_hbm.at[wid], meta_v)
    lane = lax.broadcasted_iota(jnp.int32, (16,), 0)
    a_vec = meta_v[pl.ds(0, 16)]
    e_vec = meta_v[pl.ds(16, 16)]
    q_vec = meta_v[pl.ds(32, 16)]

    blk0 = a_vec >> 7
    last = (e_vec - 1) >> 7
    nblk = last - blk0 + 1
    rounds = nblk[0]
    for l in range(1, 16):
        rounds = jnp.maximum(rounds, nblk[l])

    neg16 = jnp.full((16,), NEG, jnp.float32)
    zero16 = jnp.zeros((16,), jnp.int32)
    for b in range(BAGS_PER_W):
        vb_ref[pl.ds(b * 16, 16)] = neg16
        va_ref[pl.ds(b * 16, 16)] = zero16

    def round_body(r, carry):
        blk = jnp.minimum(blk0 + r, last)
        pltpu.async_copy(pt_hbm.at[q_vec * (N // 128) + blk], blocks_v,
                         sem).wait()
        for b in range(BAGS_PER_W):
            rowbase = blk[b] * 128
            a_s = a_vec[b]
            e_s = e_vec[b]
            vb = vb_ref[pl.ds(b * 16, 16)]
            va = va_ref[pl.ds(b * 16, 16)]
            for j in range(8):
                sv = blocks_v[b, pl.ds(j * 16, 16)]
                rid = rowbase + j * 16 + lane
                ok = (rid >= a_s) & (rid < e_s)
                val = jnp.where(ok, sv, NEG)
                upd = val > vb
                vb = jnp.where(upd, val, vb)
                va = jnp.where(upd, rid, va)
            vb_ref[pl.ds(b * 16, 16)] = vb
            va_ref[pl.ds(b * 16, 16)] = va
        return carry

    lax.fori_loop(0, rounds, round_body, jnp.int32(0))

    jsv = zero16
    for b in range(BAGS_PER_W):
        vb = vb_ref[pl.ds(b * 16, 16)]
        va = va_ref[pl.ds(b * 16, 16)]
        best = vb[0]
        barg = va[0]
        for l in range(1, 16):
            k = vb[l]
            r = va[l]
            upd = (k > best) | ((k == best) & (r < barg))
            best = jnp.where(upd, k, best)
            barg = jnp.where(upd, r, barg)
        jsv = jnp.where(lane == b, barg, jsv)

    pltpu.async_copy(a_hbm.at[jsv], outrows_v, sem).wait()
    pltpu.sync_copy(outrows_v, out_hbm.at[pl.ds(base, BAGS_PER_W)])


@functools.lru_cache(maxsize=1)
def _sc_select():
    return pl.kernel(
        _sc_body,
        out_type=jax.ShapeDtypeStruct((B, LANES), jnp.float32),
        mesh=plsc.VectorSubcoreMesh(core_axis_name="c", subcore_axis_name="s",
                                    num_cores=NUM_CORES,
                                    num_subcores=NUM_SUBCORES),
        scratch_types=[
            pltpu.VMEM((3 * BAGS_PER_W,), jnp.int32),
            pltpu.VMEM((BAGS_PER_W, 128), jnp.float32),
            pltpu.VMEM((BAGS_PER_W, LANES), jnp.float32),
            pltpu.VMEM((BAGS_PER_W * 16,), jnp.float32),
            pltpu.VMEM((BAGS_PER_W * 16,), jnp.int32),
            pltpu.SemaphoreType.DMA,
        ],
    )


def _tc_logits(x, w_pad, bias_pad):
    return pl.pallas_call(
        _tc_body,
        grid=(N // ROW_BLK,),
        in_specs=[
            pl.BlockSpec((ROW_BLK, HIDDEN), lambda i: (i, 0)),
            pl.BlockSpec((HIDDEN, LANES), lambda i: (0, 0)),
            pl.BlockSpec((1, LANES), lambda i: (0, 0)),
        ],
        out_specs=[
            pl.BlockSpec((ROW_BLK, LANES), lambda i: (i, 0)),
            pl.BlockSpec((QROWS, ROW_BLK // 128, 128), lambda i: (0, i, 0)),
        ],
        out_shape=[
            jax.ShapeDtypeStruct((N, LANES), jnp.float32),
            jax.ShapeDtypeStruct((QROWS, N // 128, 128), jnp.float32),
        ],
    )(x, w_pad, bias_pad)


def kernel(x, scope, query, rel_mat, bias):
    w_pad = jnp.zeros((HIDDEN, LANES), jnp.float32).at[:, :REL].set(rel_mat)
    bias_pad = jnp.full((LANES,), -1e30, jnp.float32).at[:REL].set(bias)
    A, PT = _tc_logits(x, w_pad, bias_pad.reshape(1, LANES))
    scope32 = scope.astype(jnp.int32)
    meta = jnp.concatenate([
        scope32[:, 0].reshape(NUM_WORKERS, BAGS_PER_W),
        scope32[:, 1].reshape(NUM_WORKERS, BAGS_PER_W),
        query.astype(jnp.int32).reshape(NUM_WORKERS, BAGS_PER_W),
    ], axis=1)
    out_full = _sc_select()(PT.reshape(QROWS * (N // 128), 128), A, meta)
    return out_full[:, :REL], rel_mat


# speedup vs baseline: 1.0732x; 1.0634x over previous
import functools

import jax
import jax.numpy as jnp
from jax import lax
from jax.experimental import pallas as pl
from jax.experimental.pallas import tpu as pltpu
from jax.experimental.pallas import tpu_sc as plsc

N = 8192
HIDDEN = 768
REL = 53
B = 512
LANES = 128
NUM_CORES = 2
NUM_SUBCORES = 16
NUM_WORKERS = NUM_CORES * NUM_SUBCORES
BAGS_PER_W = B // NUM_WORKERS
ROW_BLK = 2048
QROWS = 64
NEG = -3.0e38


def _tc_body(x_ref, w_ref, b_ref, a_ref, pt_ref):
    xb = x_ref[...]
    w = jnp.concatenate(
        [w_ref[...], jnp.zeros((HIDDEN, LANES - REL), jnp.float32)], axis=1)
    bias = jnp.concatenate(
        [b_ref[...], jnp.full((1, LANES - REL), -1e30, jnp.float32)], axis=1)
    logits = lax.dot_general(
        xb, w, (((1,), (0,)), ((), ())),
        precision=None,
        preferred_element_type=jnp.float32,
    ) + bias
    m = jnp.max(logits, axis=1, keepdims=True)
    u = jnp.exp(logits - m)
    probs = u / jnp.sum(u, axis=1, keepdims=True)
    a_ref[...] = logits
    for k in range(ROW_BLK // 128):
        pt_ref[:, k, :] = probs[k * 128:(k + 1) * 128, :QROWS].T


def _sc_body(pt_hbm, a_hbm, meta_hbm, out_hbm,
             meta_v, blocks_v, outrows_v, vb_ref, va_ref, sem):
    wid = lax.axis_index("s") * NUM_CORES + lax.axis_index("c")
    base = wid * BAGS_PER_W
    pltpu.sync_copy(meta_hbm.at[wid], meta_v)
    lane = lax.broadcasted_iota(jnp.int32, (16,), 0)
    a_vec = meta_v[pl.ds(0, 16)]
    e_vec = meta_v[pl.ds(16, 16)]
    q_vec = meta_v[pl.ds(32, 16)]

    blk0 = a_vec >> 7
    last = (e_vec - 1) >> 7
    nblk = last - blk0 + 1
    rounds = nblk[0]
    for l in range(1, 16):
        rounds = jnp.maximum(rounds, nblk[l])

    neg16 = jnp.full((16,), NEG, jnp.float32)
    zero16 = jnp.zeros((16,), jnp.int32)
    for b in range(BAGS_PER_W):
        vb_ref[pl.ds(b * 16, 16)] = neg16
        va_ref[pl.ds(b * 16, 16)] = zero16

    def round_body(r, carry):
        blk = jnp.minimum(blk0 + r, last)
        pltpu.async_copy(pt_hbm.at[q_vec * (N // 128) + blk], blocks_v,
                         sem).wait()
        for b in range(BAGS_PER_W):
            rowbase = blk[b] * 128
            a_s = a_vec[b]
            e_s = e_vec[b]
            vb = vb_ref[pl.ds(b * 16, 16)]
            va = va_ref[pl.ds(b * 16, 16)]
            for j in range(8):
                sv = blocks_v[b, pl.ds(j * 16, 16)]
                rid = rowbase + j * 16 + lane
                ok = (rid >= a_s) & (rid < e_s)
                val = jnp.where(ok, sv, NEG)
                upd = val > vb
                vb = jnp.where(upd, val, vb)
                va = jnp.where(upd, rid, va)
            vb_ref[pl.ds(b * 16, 16)] = vb
            va_ref[pl.ds(b * 16, 16)] = va
        return carry

    lax.fori_loop(0, rounds, round_body, jnp.int32(0))

    jsv = zero16
    for b in range(BAGS_PER_W):
        vb = vb_ref[pl.ds(b * 16, 16)]
        va = va_ref[pl.ds(b * 16, 16)]
        best = vb[0]
        barg = va[0]
        for l in range(1, 16):
            k = vb[l]
            r = va[l]
            upd = (k > best) | ((k == best) & (r < barg))
            best = jnp.where(upd, k, best)
            barg = jnp.where(upd, r, barg)
        jsv = jnp.where(lane == b, barg, jsv)

    pltpu.async_copy(a_hbm.at[jsv], outrows_v, sem).wait()
    pltpu.sync_copy(outrows_v, out_hbm.at[pl.ds(base, BAGS_PER_W)])


@functools.lru_cache(maxsize=1)
def _sc_select():
    return pl.kernel(
        _sc_body,
        out_type=jax.ShapeDtypeStruct((B, LANES), jnp.float32),
        mesh=plsc.VectorSubcoreMesh(core_axis_name="c", subcore_axis_name="s",
                                    num_cores=NUM_CORES,
                                    num_subcores=NUM_SUBCORES),
        scratch_types=[
            pltpu.VMEM((3 * BAGS_PER_W,), jnp.int32),
            pltpu.VMEM((BAGS_PER_W, 128), jnp.float32),
            pltpu.VMEM((BAGS_PER_W, LANES), jnp.float32),
            pltpu.VMEM((BAGS_PER_W * 16,), jnp.float32),
            pltpu.VMEM((BAGS_PER_W * 16,), jnp.int32),
            pltpu.SemaphoreType.DMA,
        ],
    )


def _tc_logits(x, w_pad, bias_pad):
    return pl.pallas_call(
        _tc_body,
        grid=(N // ROW_BLK,),
        in_specs=[
            pl.BlockSpec((ROW_BLK, HIDDEN), lambda i: (i, 0)),
            pl.BlockSpec((HIDDEN, REL), lambda i: (0, 0)),
            pl.BlockSpec((1, REL), lambda i: (0, 0)),
        ],
        out_specs=[
            pl.BlockSpec((ROW_BLK, LANES), lambda i: (i, 0)),
            pl.BlockSpec((QROWS, ROW_BLK // 128, 128), lambda i: (0, i, 0)),
        ],
        out_shape=[
            jax.ShapeDtypeStruct((N, LANES), jnp.float32),
            jax.ShapeDtypeStruct((QROWS, N // 128, 128), jnp.float32),
        ],
    )(x, w_pad, bias_pad)


def kernel(x, scope, query, rel_mat, bias):
    A, PT = _tc_logits(x, rel_mat, bias.reshape(1, REL))
    scope32 = scope.astype(jnp.int32)
    packed = jnp.concatenate([
        scope32[:, 0].reshape(NUM_WORKERS, BAGS_PER_W),
        scope32[:, 1].reshape(NUM_WORKERS, BAGS_PER_W),
        query.astype(jnp.int32).reshape(NUM_WORKERS, BAGS_PER_W),
    ], axis=1)
    out_full = _sc_select()(PT.reshape(QROWS * (N // 128), 128), A, packed)
    return out_full[:, :REL], rel_mat


# speedup vs baseline: 1.0951x; 1.0204x over previous
import functools

import jax
import jax.numpy as jnp
from jax import lax
from jax.experimental import pallas as pl
from jax.experimental.pallas import tpu as pltpu
from jax.experimental.pallas import tpu_sc as plsc

N = 8192
HIDDEN = 768
REL = 53
B = 512
LANES = 128
NUM_CORES = 2
NUM_SUBCORES = 16
NUM_WORKERS = NUM_CORES * NUM_SUBCORES
BAGS_PER_W = B // NUM_WORKERS
ROW_BLK = 2048
QROWS = 64
NEG = -3.0e38


def _tc_body(x_ref, w_ref, b_ref, a_ref, pt_ref):
    xb = x_ref[...]
    w = jnp.concatenate(
        [w_ref[...], jnp.zeros((HIDDEN, LANES - REL), jnp.float32)], axis=1)
    bias = jnp.concatenate(
        [b_ref[...], jnp.full((1, LANES - REL), -1e30, jnp.float32)], axis=1)
    logits = lax.dot_general(
        xb, w, (((1,), (0,)), ((), ())),
        precision=None,
        preferred_element_type=jnp.float32,
    ) + bias
    m = jnp.max(logits, axis=1, keepdims=True)
    u = jnp.exp(logits - m)
    probs = u / jnp.sum(u, axis=1, keepdims=True)
    a_ref[...] = logits
    for k in range(ROW_BLK // 128):
        pt_ref[:, k, :] = probs[k * 128:(k + 1) * 128, :QROWS].T


def _sc_body(pt_hbm, a_hbm, scope_hbm, query_hbm, out_hbm,
             meta_v, blocks_v, outrows_v, vb_ref, va_ref, sem):
    wid = lax.axis_index("s") * NUM_CORES + lax.axis_index("c")
    base = wid * BAGS_PER_W
    h1 = pltpu.async_copy(scope_hbm.at[pl.ds(base * 2, 2 * BAGS_PER_W)],
                          meta_v.at[pl.ds(0, 2 * BAGS_PER_W)], sem)
    h2 = pltpu.async_copy(query_hbm.at[pl.ds(base, BAGS_PER_W)],
                          meta_v.at[pl.ds(2 * BAGS_PER_W, BAGS_PER_W)], sem)
    h1.wait()
    h2.wait()
    lane = lax.broadcasted_iota(jnp.int32, (16,), 0)
    iv0 = meta_v[pl.ds(0, 16)]
    iv1 = meta_v[pl.ds(16, 16)]
    q_vec = meta_v[pl.ds(32, 16)]
    a_vec = jnp.zeros((16,), jnp.int32)
    e_vec = jnp.zeros((16,), jnp.int32)
    for b in range(BAGS_PER_W):
        iv = iv0 if 2 * b < 16 else iv1
        a_vec = jnp.where(lane == b, iv[(2 * b) % 16], a_vec)
        e_vec = jnp.where(lane == b, iv[(2 * b + 1) % 16], e_vec)

    blk0 = a_vec >> 7
    last = (e_vec - 1) >> 7
    nblk = last - blk0 + 1
    rounds = nblk[0]
    for l in range(1, 16):
        rounds = jnp.maximum(rounds, nblk[l])

    neg16 = jnp.full((16,), NEG, jnp.float32)
    zero16 = jnp.zeros((16,), jnp.int32)
    for b in range(BAGS_PER_W):
        vb_ref[pl.ds(b * 16, 16)] = neg16
        va_ref[pl.ds(b * 16, 16)] = zero16

    def round_body(r, carry):
        blk = jnp.minimum(blk0 + r, last)
        pltpu.async_copy(pt_hbm.at[q_vec * (N // 128) + blk], blocks_v,
                         sem).wait()
        for b in range(BAGS_PER_W):
            rowbase = blk[b] * 128
            a_s = a_vec[b]
            e_s = e_vec[b]
            vb = vb_ref[pl.ds(b * 16, 16)]
            va = va_ref[pl.ds(b * 16, 16)]
            for j in range(8):
                sv = blocks_v[b, pl.ds(j * 16, 16)]
                rid = rowbase + j * 16 + lane
                ok = (rid >= a_s) & (rid < e_s)
                val = jnp.where(ok, sv, NEG)
                upd = val > vb
                vb = jnp.where(upd, val, vb)
                va = jnp.where(upd, rid, va)
            vb_ref[pl.ds(b * 16, 16)] = vb
            va_ref[pl.ds(b * 16, 16)] = va
        return carry

    lax.fori_loop(0, rounds, round_body, jnp.int32(0))

    jsv = zero16
    for b in range(BAGS_PER_W):
        vb = vb_ref[pl.ds(b * 16, 16)]
        va = va_ref[pl.ds(b * 16, 16)]
        best = vb[0]
        barg = va[0]
        for l in range(1, 16):
            k = vb[l]
            r = va[l]
            upd = (k > best) | ((k == best) & (r < barg))
            best = jnp.where(upd, k, best)
            barg = jnp.where(upd, r, barg)
        jsv = jnp.where(lane == b, barg, jsv)

    pltpu.async_copy(a_hbm.at[jsv], outrows_v, sem).wait()
    pltpu.sync_copy(outrows_v, out_hbm.at[pl.ds(base, BAGS_PER_W)])


@functools.lru_cache(maxsize=1)
def _sc_select():
    return pl.kernel(
        _sc_body,
        out_type=jax.ShapeDtypeStruct((B, LANES), jnp.float32),
        mesh=plsc.VectorSubcoreMesh(core_axis_name="c", subcore_axis_name="s",
                                    num_cores=NUM_CORES,
                                    num_subcores=NUM_SUBCORES),
        scratch_types=[
            pltpu.VMEM((3 * BAGS_PER_W,), jnp.int32),
            pltpu.VMEM((BAGS_PER_W, 128), jnp.float32),
            pltpu.VMEM((BAGS_PER_W, LANES), jnp.float32),
            pltpu.VMEM((BAGS_PER_W * 16,), jnp.float32),
            pltpu.VMEM((BAGS_PER_W * 16,), jnp.int32),
            pltpu.SemaphoreType.DMA,
        ],
    )


def _tc_logits(x, w_pad, bias_pad):
    return pl.pallas_call(
        _tc_body,
        grid=(N // ROW_BLK,),
        in_specs=[
            pl.BlockSpec((ROW_BLK, HIDDEN), lambda i: (i, 0)),
            pl.BlockSpec((HIDDEN, REL), lambda i: (0, 0)),
            pl.BlockSpec((1, REL), lambda i: (0, 0)),
        ],
        out_specs=[
            pl.BlockSpec((ROW_BLK, LANES), lambda i: (i, 0)),
            pl.BlockSpec((QROWS, ROW_BLK // 128, 128), lambda i: (0, i, 0)),
        ],
        out_shape=[
            jax.ShapeDtypeStruct((N, LANES), jnp.float32),
            jax.ShapeDtypeStruct((QROWS, N // 128, 128), jnp.float32),
        ],
    )(x, w_pad, bias_pad)


def kernel(x, scope, query, rel_mat, bias):
    A, PT = _tc_logits(x, rel_mat, bias.reshape(1, REL))
    scope32 = scope.astype(jnp.int32)
    out_full = _sc_select()(PT.reshape(QROWS * (N // 128), 128), A,
                            scope32.reshape(-1), query.astype(jnp.int32))
    return out_full[:, :REL], rel_mat
